# Initial kernel scaffold; baseline (speedup 1.0000x reference)
import jax
import jax.numpy as jnp
from jax.experimental import pallas as pl


def kernel(inputs, L_rows, L_cols, L_vals, W1_l0, b1_l0, W2_l0, b2_l0, W3_l0, b3_l0, W4_l0, b4_l0, gamma_l0, beta_l0, W1_l1, b1_l1, W2_l1, b2_l1, W3_l1, b3_l1, W4_l1, b4_l1, gamma_l1, beta_l1, fc1_W, fc1_b, fc2_W, fc2_b):
    raise NotImplementedError("write your pallas kernel here")


# speedup vs baseline: 4.1422x; 4.1422x over previous
import dataclasses
import functools

import jax
import jax.numpy as jnp
from jax import lax
from jax.experimental import pallas as pl
from jax.experimental.pallas import tpu as pltpu
from jax.experimental.pallas import tpu_sc as plsc

M = 10000
W_IN = 128
NNZ = M * 16
B = 4
HID = 32
FC1 = 256
NCLS = 53

NC = 2
NS = 16
CH = (B * HID) // NC
EK = 128
CHUNKS = -(-NNZ // (NS * EK))
EPW = CHUNKS * EK
NNZ_PAD = EPW * NS
RPT = 632
RPT_LAST = M - (NS - 1) * RPT

_sc_mesh = plsc.VectorSubcoreMesh(core_axis_name="c", subcore_axis_name="s")


def _spmm_body(x_hbm, init_hbm, rows_hbm, cols_hbm, vals_hbm, out_hbm,
               acc_sh, rows_v, cols_v, vals_v, g_v, sem):
  c = lax.axis_index("c")
  s = lax.axis_index("s")

  row0 = pl.multiple_of(s * RPT, 8)

  @pl.when(s < NS - 1)
  def _():
    pltpu.sync_copy(init_hbm.at[c, pl.ds(row0, RPT)],
                    acc_sh.at[pl.ds(row0, RPT)])

  @pl.when(s == NS - 1)
  def _():
    pltpu.sync_copy(init_hbm.at[c, pl.ds((NS - 1) * RPT, RPT_LAST)],
                    acc_sh.at[pl.ds((NS - 1) * RPT, RPT_LAST)])

  plsc.subcore_barrier()

  @pl.loop(0, CHUNKS)
  def _edge_chunk(i):
    base = pl.multiple_of(s * EPW + i * EK, 8)
    pltpu.sync_copy(rows_hbm.at[pl.ds(base, EK)], rows_v)
    pltpu.sync_copy(cols_hbm.at[pl.ds(base, EK)], cols_v)
    pltpu.sync_copy(vals_hbm.at[pl.ds(base, EK)], vals_v)
    pltpu.async_copy(x_hbm.at[c].at[cols_v], g_v, sem).wait()

    @pl.loop(0, EK)
    def _scale(e):
      bv = plsc.load_gather(vals_v, [jnp.full((16,), e, jnp.int32)])
      for k in range(CH // 16):
        g_v[e, pl.ds(k * 16, 16)] = g_v[e, pl.ds(k * 16, 16)] * bv

    pltpu.sync_copy(g_v, acc_sh.at[rows_v], add=True)

  plsc.subcore_barrier()

  @pl.when(s < NS - 1)
  def _():
    pltpu.sync_copy(acc_sh.at[pl.ds(row0, RPT)],
                    out_hbm.at[c, pl.ds(row0, RPT)])

  @pl.when(s == NS - 1)
  def _():
    pltpu.sync_copy(acc_sh.at[pl.ds((NS - 1) * RPT, RPT_LAST)],
                    out_hbm.at[c, pl.ds((NS - 1) * RPT, RPT_LAST)])


_sc_params = pltpu.CompilerParams()
if "needs_layout_passes" in pltpu.CompilerParams.__dataclass_fields__:
  _sc_params = dataclasses.replace(_sc_params, needs_layout_passes=False)
if "use_tc_tiling_on_sc" in pltpu.CompilerParams.__dataclass_fields__:
  _sc_params = dataclasses.replace(_sc_params, use_tc_tiling_on_sc=False)

_spmm_call = pl.kernel(
    _spmm_body,
    compiler_params=_sc_params,
    out_type=jax.ShapeDtypeStruct((NC, M, CH), jnp.float32),
    mesh=_sc_mesh,
    scratch_types=[
        pltpu.VMEM_SHARED((M, CH), jnp.float32),
        pltpu.VMEM((EK,), jnp.int32),
        pltpu.VMEM((EK,), jnp.int32),
        pltpu.VMEM((EK,), jnp.float32),
        pltpu.VMEM((EK, CH), jnp.float32),
        pltpu.SemaphoreType.DMA,
    ],
)


def _spmm_add(x, init, rows, cols, vals):
  return _spmm_call(x, init, rows, cols, vals)


_PBM = 1000


def _proj0_body(x_ref, w_ref, p1_ref, p2_ref, p3_ref, p4_ref):
  prefs = [p1_ref, p2_ref, p3_ref, p4_ref]
  w = w_ref[...]
  for b in range(B):
    pb = jnp.dot(x_ref[b], w, preferred_element_type=jnp.float32)
    ci, ii = b // 2, b % 2
    for j in range(4):
      prefs[j][ci, :, ii * HID:(ii + 1) * HID] = pb[:, j * HID:(j + 1) * HID]


def _proj0(x, wstack):
  out = jax.ShapeDtypeStruct((NC, M, CH), jnp.float32)
  pspec = pl.BlockSpec((NC, _PBM, CH), lambda i: (0, i, 0))
  return pl.pallas_call(
      _proj0_body,
      grid=(M // _PBM,),
      in_specs=[
          pl.BlockSpec((B, _PBM, W_IN), lambda i: (0, i, 0)),
          pl.BlockSpec((W_IN, 4 * HID), lambda i: (0, 0)),
      ],
      out_specs=[pspec, pspec, pspec, pspec],
      out_shape=[out, out, out, out],
  )(x, wstack)


def _bn_stats(y2):
  n = float(NC * M * 2)
  s64 = jnp.sum(y2, axis=0)
  q64 = jnp.sum(y2 * y2, axis=0)
  s32 = s64[:HID] + s64[HID:]
  q32 = q64[:HID] + q64[HID:]
  mean = s32 / n
  var = q32 / n - mean * mean
  return mean, var


def _bn_scale_off(mean, var, gamma, beta):
  inv = gamma / jnp.sqrt(var + 1e-5)
  scale = jnp.concatenate([inv, inv])
  off = jnp.concatenate([beta - mean * inv, beta - mean * inv])
  return scale, off


def _bn_proj1_body(y_ref, g_ref, b_ref, w_ref, p1_ref, p2_ref, p3_ref, p4_ref):
  y2 = y_ref[...].reshape(NC * M, CH)
  mean, var = _bn_stats(y2)
  scale, off = _bn_scale_off(mean, var, g_ref[0], b_ref[0])
  act = jnp.maximum(y2 * scale[None, :] + off[None, :], 0.0)
  pall = jnp.dot(act, w_ref[...], preferred_element_type=jnp.float32)
  pall = pall.reshape(NC, M, 4 * CH)
  prefs = [p1_ref, p2_ref, p3_ref, p4_ref]
  for j in range(4):
    prefs[j][...] = pall[:, :, j * CH:(j + 1) * CH]


def _bn_proj1(y, gamma, beta, w64):
  out = jax.ShapeDtypeStruct((NC, M, CH), jnp.float32)
  return pl.pallas_call(
      _bn_proj1_body,
      out_shape=[out, out, out, out],
  )(y, gamma.reshape(1, HID), beta.reshape(1, HID), w64)


def _bn_act_body(y_ref, g_ref, b_ref, a_ref):
  y2 = y_ref[...].reshape(NC * M, CH)
  mean, var = _bn_stats(y2)
  scale, off = _bn_scale_off(mean, var, g_ref[0], b_ref[0])
  act = jnp.maximum(y2 * scale[None, :] + off[None, :], 0.0)
  a_ref[...] = act.reshape(NC, M, CH)


def _bn_act(y, gamma, beta):
  return pl.pallas_call(
      _bn_act_body,
      out_shape=jax.ShapeDtypeStruct((NC, M, CH), jnp.float32),
  )(y, gamma.reshape(1, HID), beta.reshape(1, HID))


_FCM = 200
_FCK = _FCM * HID
_FCN = M // _FCM


def _fc_body(a_ref, w_ref, b1_ref, w2_ref, b2_ref, o_ref, acc_ref):
  i = pl.program_id(0)

  @pl.when(i == 0)
  def _():
    acc_ref[...] = jnp.zeros_like(acc_ref)

  a = a_ref[...].reshape(NC, _FCM, 2, HID)
  a = jnp.transpose(a, (0, 2, 1, 3)).reshape(B, _FCK)
  a = jnp.concatenate([a, jnp.zeros((4, _FCK), jnp.float32)], axis=0)
  acc_ref[...] += jnp.dot(a.astype(jnp.bfloat16),
                          w_ref[...].astype(jnp.bfloat16),
                          preferred_element_type=jnp.float32)

  @pl.when(i == _FCN - 1)
  def _():
    h = jnp.maximum(acc_ref[...][:B] + b1_ref[...], 0.0)
    o_ref[...] = jnp.dot(h, w2_ref[...],
                         preferred_element_type=jnp.float32) + b2_ref[...]


def _fc_head(act, fc1_w, fc1_b, fc2_w, fc2_b):
  return pl.pallas_call(
      _fc_body,
      grid=(_FCN,),
      in_specs=[
          pl.BlockSpec((NC, _FCM, CH), lambda i: (0, i, 0)),
          pl.BlockSpec((_FCK, FC1), lambda i: (i, 0)),
          pl.BlockSpec((1, FC1), lambda i: (0, 0)),
          pl.BlockSpec((FC1, NCLS), lambda i: (0, 0)),
          pl.BlockSpec((1, NCLS), lambda i: (0, 0)),
      ],
      out_specs=pl.BlockSpec((B, NCLS), lambda i: (0, 0)),
      out_shape=jax.ShapeDtypeStruct((B, NCLS), jnp.float32),
      scratch_shapes=[pltpu.VMEM((8, FC1), jnp.float32)],
  )(act, fc1_w, fc1_b.reshape(1, FC1), fc2_w, fc2_b.reshape(1, NCLS))


@jax.jit
def kernel(inputs, L_rows, L_cols, L_vals,
           W1_l0, b1_l0, W2_l0, b2_l0, W3_l0, b3_l0, W4_l0, b4_l0,
           gamma_l0, beta_l0,
           W1_l1, b1_l1, W2_l1, b2_l1, W3_l1, b3_l1, W4_l1, b4_l1,
           gamma_l1, beta_l1,
           fc1_W, fc1_b, fc2_W, fc2_b):
  pad = NNZ_PAD - NNZ
  spread = (jnp.arange(pad, dtype=jnp.int32) * 7) % M
  rows = jnp.concatenate([L_rows.astype(jnp.int32), spread])
  cols = jnp.concatenate([L_cols.astype(jnp.int32), spread])
  vals = jnp.concatenate([L_vals, jnp.zeros((pad,), jnp.float32)])

  wstack0 = jnp.concatenate(
      [W1_l0 - W3_l0, W2_l0 - 2.0 * W4_l0, W3_l0, W4_l0], axis=1)
  wstack1 = jnp.concatenate(
      [W1_l1 - W3_l1, W2_l1 - 2.0 * W4_l1, W3_l1, W4_l1], axis=1)
  z32 = jnp.zeros((HID, HID), jnp.float32)
  w64 = jnp.concatenate(
      [jnp.block([[wstack1[:, j * HID:(j + 1) * HID], z32],
                  [z32, wstack1[:, j * HID:(j + 1) * HID]]])
       for j in range(4)], axis=1)

  p1, p2, p3, p4 = _proj0(inputs, wstack0)
  t = _spmm_add(p4, p3, rows, cols, vals)
  t = _spmm_add(t, p2, rows, cols, vals)
  y0 = _spmm_add(t, p1, rows, cols, vals)

  q1, q2, q3, q4 = _bn_proj1(y0, gamma_l0, beta_l0, w64)
  t = _spmm_add(q4, q3, rows, cols, vals)
  t = _spmm_add(t, q2, rows, cols, vals)
  y1 = _spmm_add(t, q1, rows, cols, vals)

  act = _bn_act(y1, gamma_l1, beta_l1)
  return _fc_head(act, fc1_W, fc1_b, fc2_W, fc2_b)


# speedup vs baseline: 5.5978x; 1.3514x over previous
import dataclasses
import functools

import jax
import jax.numpy as jnp
from jax import lax
from jax.experimental import pallas as pl
from jax.experimental.pallas import tpu as pltpu
from jax.experimental.pallas import tpu_sc as plsc

M = 10000
W_IN = 128
NNZ = M * 16
B = 4
HID = 32
FC1 = 256
NCLS = 53

NC = 2
NS = 16
CH = (B * HID) // NC
EK = 128
NB = 2
CHUNKS = 80
EPW = CHUNKS * EK
NNZ_PAD = EPW * NS
RPT = 632
RPT_LAST = M - (NS - 1) * RPT

_sc_mesh = plsc.VectorSubcoreMesh(core_axis_name="c", subcore_axis_name="s")


def _spmm_body(x_hbm, init_hbm, rows_hbm, cols_hbm, vals_hbm, out_hbm,
               acc_sh, rows_v, cols_v, vals_v,
               g0, g1, s0, s1, gsem0, gsem1, ssem0, ssem1):
  c = lax.axis_index("c")
  s = lax.axis_index("s")
  gbuf = [g0, g1]
  sbuf = [s0, s1]
  gsem = [gsem0, gsem1]
  ssem = [ssem0, ssem1]

  row0 = pl.multiple_of(s * RPT, 8)

  @pl.when(s < NS - 1)
  def _():
    pltpu.sync_copy(init_hbm.at[c, pl.ds(row0, RPT)],
                    acc_sh.at[pl.ds(row0, RPT)])

  @pl.when(s == NS - 1)
  def _():
    pltpu.sync_copy(init_hbm.at[c, pl.ds((NS - 1) * RPT, RPT_LAST)],
                    acc_sh.at[pl.ds((NS - 1) * RPT, RPT_LAST)])

  pltpu.sync_copy(rows_hbm.at[s], rows_v)
  pltpu.sync_copy(cols_hbm.at[s], cols_v)
  pltpu.sync_copy(vals_hbm.at[s], vals_v)
  plsc.subcore_barrier()

  for b in range(NB):
    pltpu.async_copy(x_hbm.at[c].at[cols_v.at[b]], gbuf[b], gsem[b])

  @pl.loop(0, CHUNKS, step=NB)
  def _edge_chunk(i):
    for b in range(NB):
      ii = i + b
      pltpu.make_async_copy(x_hbm.at[c].at[cols_v.at[ii]],
                            gbuf[b], gsem[b]).wait()
      @pl.when(ii >= NB)
      def _():
        pltpu.make_async_copy(sbuf[b], acc_sh.at[rows_v.at[ii - NB]],
                              ssem[b]).wait()

      @pl.loop(0, EK)
      def _scale(e):
        bv = plsc.load_gather(
            vals_v, [jnp.full((16,), ii, jnp.int32),
                     jnp.full((16,), e, jnp.int32)])
        for k in range(CH // 16):
          sbuf[b][e, pl.ds(k * 16, 16)] = gbuf[b][e, pl.ds(k * 16, 16)] * bv

      @pl.when(ii + NB < CHUNKS)
      def _():
        pltpu.async_copy(x_hbm.at[c].at[cols_v.at[ii + NB]],
                         gbuf[b], gsem[b])
      pltpu.async_copy(sbuf[b], acc_sh.at[rows_v.at[ii]], ssem[b], add=True)

  for b in range(NB):
    pltpu.make_async_copy(sbuf[b], acc_sh.at[rows_v.at[CHUNKS - NB + b]],
                          ssem[b]).wait()
  plsc.subcore_barrier()

  @pl.when(s < NS - 1)
  def _():
    pltpu.sync_copy(acc_sh.at[pl.ds(row0, RPT)],
                    out_hbm.at[c, pl.ds(row0, RPT)])

  @pl.when(s == NS - 1)
  def _():
    pltpu.sync_copy(acc_sh.at[pl.ds((NS - 1) * RPT, RPT_LAST)],
                    out_hbm.at[c, pl.ds((NS - 1) * RPT, RPT_LAST)])


_sc_params = pltpu.CompilerParams()
if "needs_layout_passes" in pltpu.CompilerParams.__dataclass_fields__:
  _sc_params = dataclasses.replace(_sc_params, needs_layout_passes=False)
if "use_tc_tiling_on_sc" in pltpu.CompilerParams.__dataclass_fields__:
  _sc_params = dataclasses.replace(_sc_params, use_tc_tiling_on_sc=False)

_spmm_call = pl.kernel(
    _spmm_body,
    compiler_params=_sc_params,
    out_type=jax.ShapeDtypeStruct((NC, M, CH), jnp.float32),
    mesh=_sc_mesh,
    scratch_types=(
        [pltpu.VMEM_SHARED((M, CH), jnp.float32),
         pltpu.VMEM((CHUNKS, EK), jnp.int32),
         pltpu.VMEM((CHUNKS, EK), jnp.int32),
         pltpu.VMEM((CHUNKS, EK), jnp.float32)]
        + [pltpu.VMEM((EK, CH), jnp.float32)] * (2 * NB)
        + [pltpu.SemaphoreType.DMA] * (2 * NB)
    ),
)


def _spmm_add(x, init, rows, cols, vals):
  return _spmm_call(x, init, rows, cols, vals)


_PBM = 1000


def _proj0_body(x_ref, w_ref, p1_ref, p2_ref, p3_ref, p4_ref):
  prefs = [p1_ref, p2_ref, p3_ref, p4_ref]
  w = w_ref[...]
  for b in range(B):
    pb = jnp.dot(x_ref[b], w, preferred_element_type=jnp.float32)
    ci, ii = b // 2, b % 2
    for j in range(4):
      prefs[j][ci, :, ii * HID:(ii + 1) * HID] = pb[:, j * HID:(j + 1) * HID]


def _proj0(x, wstack):
  out = jax.ShapeDtypeStruct((NC, M, CH), jnp.float32)
  pspec = pl.BlockSpec((NC, _PBM, CH), lambda i: (0, i, 0))
  return pl.pallas_call(
      _proj0_body,
      grid=(M // _PBM,),
      in_specs=[
          pl.BlockSpec((B, _PBM, W_IN), lambda i: (0, i, 0)),
          pl.BlockSpec((W_IN, 4 * HID), lambda i: (0, 0)),
      ],
      out_specs=[pspec, pspec, pspec, pspec],
      out_shape=[out, out, out, out],
  )(x, wstack)


def _bn_stats(y2):
  n = float(NC * M * 2)
  s64 = jnp.sum(y2, axis=0)
  q64 = jnp.sum(y2 * y2, axis=0)
  s32 = s64[:HID] + s64[HID:]
  q32 = q64[:HID] + q64[HID:]
  mean = s32 / n
  var = q32 / n - mean * mean
  return mean, var


def _bn_scale_off(mean, var, gamma, beta):
  inv = gamma / jnp.sqrt(var + 1e-5)
  scale = jnp.concatenate([inv, inv])
  off = jnp.concatenate([beta - mean * inv, beta - mean * inv])
  return scale, off


def _bn_proj1_body(y_ref, g_ref, b_ref, w_ref, p1_ref, p2_ref, p3_ref, p4_ref):
  y2 = y_ref[...].reshape(NC * M, CH)
  mean, var = _bn_stats(y2)
  scale, off = _bn_scale_off(mean, var, g_ref[0], b_ref[0])
  act = jnp.maximum(y2 * scale[None, :] + off[None, :], 0.0)
  pall = jnp.dot(act, w_ref[...], preferred_element_type=jnp.float32)
  pall = pall.reshape(NC, M, 4 * CH)
  prefs = [p1_ref, p2_ref, p3_ref, p4_ref]
  for j in range(4):
    prefs[j][...] = pall[:, :, j * CH:(j + 1) * CH]


def _bn_proj1(y, gamma, beta, w64):
  out = jax.ShapeDtypeStruct((NC, M, CH), jnp.float32)
  return pl.pallas_call(
      _bn_proj1_body,
      out_shape=[out, out, out, out],
  )(y, gamma.reshape(1, HID), beta.reshape(1, HID), w64)


def _bn_act_body(y_ref, g_ref, b_ref, a_ref):
  y2 = y_ref[...].reshape(NC * M, CH)
  mean, var = _bn_stats(y2)
  scale, off = _bn_scale_off(mean, var, g_ref[0], b_ref[0])
  act = jnp.maximum(y2 * scale[None, :] + off[None, :], 0.0)
  a_ref[...] = act.reshape(NC, M, CH)


def _bn_act(y, gamma, beta):
  return pl.pallas_call(
      _bn_act_body,
      out_shape=jax.ShapeDtypeStruct((NC, M, CH), jnp.float32),
  )(y, gamma.reshape(1, HID), beta.reshape(1, HID))


_FCM = 200
_FCK = _FCM * HID
_FCN = M // _FCM


def _fc_body(a_ref, w_ref, b1_ref, w2_ref, b2_ref, o_ref, acc_ref):
  i = pl.program_id(0)

  @pl.when(i == 0)
  def _():
    acc_ref[...] = jnp.zeros_like(acc_ref)

  a = a_ref[...].reshape(NC, _FCM, 2, HID)
  a = jnp.transpose(a, (0, 2, 1, 3)).reshape(B, _FCK)
  a = jnp.concatenate([a, jnp.zeros((4, _FCK), jnp.float32)], axis=0)
  acc_ref[...] += jnp.dot(a.astype(jnp.bfloat16),
                          w_ref[...].astype(jnp.bfloat16),
                          preferred_element_type=jnp.float32)

  @pl.when(i == _FCN - 1)
  def _():
    h = jnp.maximum(acc_ref[...][:B] + b1_ref[...], 0.0)
    o_ref[...] = jnp.dot(h, w2_ref[...],
                         preferred_element_type=jnp.float32) + b2_ref[...]


def _fc_head(act, fc1_w, fc1_b, fc2_w, fc2_b):
  return pl.pallas_call(
      _fc_body,
      grid=(_FCN,),
      in_specs=[
          pl.BlockSpec((NC, _FCM, CH), lambda i: (0, i, 0)),
          pl.BlockSpec((_FCK, FC1), lambda i: (i, 0)),
          pl.BlockSpec((1, FC1), lambda i: (0, 0)),
          pl.BlockSpec((FC1, NCLS), lambda i: (0, 0)),
          pl.BlockSpec((1, NCLS), lambda i: (0, 0)),
      ],
      out_specs=pl.BlockSpec((B, NCLS), lambda i: (0, 0)),
      out_shape=jax.ShapeDtypeStruct((B, NCLS), jnp.float32),
      scratch_shapes=[pltpu.VMEM((8, FC1), jnp.float32)],
  )(act, fc1_w, fc1_b.reshape(1, FC1), fc2_w, fc2_b.reshape(1, NCLS))


@jax.jit
def kernel(inputs, L_rows, L_cols, L_vals,
           W1_l0, b1_l0, W2_l0, b2_l0, W3_l0, b3_l0, W4_l0, b4_l0,
           gamma_l0, beta_l0,
           W1_l1, b1_l1, W2_l1, b2_l1, W3_l1, b3_l1, W4_l1, b4_l1,
           gamma_l1, beta_l1,
           fc1_W, fc1_b, fc2_W, fc2_b):
  pad = NNZ_PAD - NNZ
  spread = (jnp.arange(pad, dtype=jnp.int32) * 7) % M
  rows = jnp.concatenate([L_rows.astype(jnp.int32), spread])
  rows = rows.reshape(NS, CHUNKS, EK)
  cols = jnp.concatenate([L_cols.astype(jnp.int32), spread])
  cols = cols.reshape(NS, CHUNKS, EK)
  vals = jnp.concatenate([L_vals, jnp.zeros((pad,), jnp.float32)])
  vals = vals.reshape(NS, CHUNKS, EK)

  wstack0 = jnp.concatenate(
      [W1_l0 - W3_l0, W2_l0 - 2.0 * W4_l0, W3_l0, W4_l0], axis=1)
  wstack1 = jnp.concatenate(
      [W1_l1 - W3_l1, W2_l1 - 2.0 * W4_l1, W3_l1, W4_l1], axis=1)
  z32 = jnp.zeros((HID, HID), jnp.float32)
  w64 = jnp.concatenate(
      [jnp.block([[wstack1[:, j * HID:(j + 1) * HID], z32],
                  [z32, wstack1[:, j * HID:(j + 1) * HID]]])
       for j in range(4)], axis=1)

  p1, p2, p3, p4 = _proj0(inputs, wstack0)
  t = _spmm_add(p4, p3, rows, cols, vals)
  t = _spmm_add(t, p2, rows, cols, vals)
  y0 = _spmm_add(t, p1, rows, cols, vals)

  q1, q2, q3, q4 = _bn_proj1(y0, gamma_l0, beta_l0, w64)
  t = _spmm_add(q4, q3, rows, cols, vals)
  t = _spmm_add(t, q2, rows, cols, vals)
  y1 = _spmm_add(t, q1, rows, cols, vals)

  act = _bn_act(y1, gamma_l1, beta_l1)
  return _fc_head(act, fc1_W, fc1_b, fc2_W, fc2_b)


# speedup vs baseline: 10.8218x; 1.9332x over previous
import dataclasses
import functools

import jax
import jax.numpy as jnp
from jax import lax
from jax.experimental import pallas as pl
from jax.experimental.pallas import tpu as pltpu
from jax.experimental.pallas import tpu_sc as plsc

M = 10000
W_IN = 128
NNZ = M * 16
B = 4
HID = 32
FC1 = 256
NCLS = 53

NC = 2
NS = 16
CH = (B * HID) // NC
EK = 128
NB = 2
CHUNKS = 80
EPW = CHUNKS * EK
NNZ_PAD = EPW * NS
RPT = 632
RPT_LAST = M - (NS - 1) * RPT

_sc_mesh = plsc.VectorSubcoreMesh(core_axis_name="c", subcore_axis_name="s")


def _spmm_body(x_hbm, init_hbm, rows_hbm, cols_hbm, vals_hbm, out_hbm,
               acc_sh, rows_v, cols_v,
               g0, g1, s0, s1, v0, v1,
               gsem0, gsem1, ssem0, ssem1, vsem0, vsem1):
  c = lax.axis_index("c")
  s = lax.axis_index("s")
  gbuf = [g0, g1]
  sbuf = [s0, s1]
  vbuf = [v0, v1]
  gsem = [gsem0, gsem1]
  ssem = [ssem0, ssem1]
  vsem = [vsem0, vsem1]

  row0 = pl.multiple_of(s * RPT, 8)

  @pl.when(s < NS - 1)
  def _():
    pltpu.sync_copy(init_hbm.at[c, pl.ds(row0, RPT)],
                    acc_sh.at[pl.ds(row0, RPT)])

  @pl.when(s == NS - 1)
  def _():
    pltpu.sync_copy(init_hbm.at[c, pl.ds((NS - 1) * RPT, RPT_LAST)],
                    acc_sh.at[pl.ds((NS - 1) * RPT, RPT_LAST)])

  pltpu.sync_copy(rows_hbm.at[s], rows_v)
  pltpu.sync_copy(cols_hbm.at[s], cols_v)
  plsc.subcore_barrier()

  for b in range(NB):
    pltpu.async_copy(x_hbm.at[c].at[cols_v.at[b]], gbuf[b], gsem[b])
    pltpu.async_copy(vals_hbm.at[s, b], vbuf[b], vsem[b])

  @pl.loop(0, CHUNKS, step=NB)
  def _edge_chunk(i):
    for b in range(NB):
      ii = i + b
      pltpu.make_async_copy(x_hbm.at[c].at[cols_v.at[ii]],
                            gbuf[b], gsem[b]).wait()
      pltpu.make_async_copy(vals_hbm.at[s, ii], vbuf[b], vsem[b]).wait()
      @pl.when(ii >= NB)
      def _():
        pltpu.make_async_copy(sbuf[b], acc_sh.at[rows_v.at[ii - NB]],
                              ssem[b]).wait()

      gb, sb, vb = gbuf[b], sbuf[b], vbuf[b]

      @plsc.parallel_loop(0, EK, unroll=2)
      def _scale(e):
        bv = vb[e]
        for k in range(CH // 16):
          sb[e, pl.ds(k * 16, 16)] = gb[e, pl.ds(k * 16, 16)] * bv

      @pl.when(ii + NB < CHUNKS)
      def _():
        pltpu.async_copy(x_hbm.at[c].at[cols_v.at[ii + NB]],
                         gbuf[b], gsem[b])
        pltpu.async_copy(vals_hbm.at[s, ii + NB], vbuf[b], vsem[b])
      pltpu.async_copy(sbuf[b], acc_sh.at[rows_v.at[ii]], ssem[b], add=True)

  for b in range(NB):
    pltpu.make_async_copy(sbuf[b], acc_sh.at[rows_v.at[CHUNKS - NB + b]],
                          ssem[b]).wait()
  plsc.subcore_barrier()

  @pl.when(s < NS - 1)
  def _():
    pltpu.sync_copy(acc_sh.at[pl.ds(row0, RPT)],
                    out_hbm.at[c, pl.ds(row0, RPT)])

  @pl.when(s == NS - 1)
  def _():
    pltpu.sync_copy(acc_sh.at[pl.ds((NS - 1) * RPT, RPT_LAST)],
                    out_hbm.at[c, pl.ds((NS - 1) * RPT, RPT_LAST)])


_sc_params = pltpu.CompilerParams()
if "needs_layout_passes" in pltpu.CompilerParams.__dataclass_fields__:
  _sc_params = dataclasses.replace(_sc_params, needs_layout_passes=False)
if "use_tc_tiling_on_sc" in pltpu.CompilerParams.__dataclass_fields__:
  _sc_params = dataclasses.replace(_sc_params, use_tc_tiling_on_sc=False)

_spmm_call = pl.kernel(
    _spmm_body,
    compiler_params=_sc_params,
    out_type=jax.ShapeDtypeStruct((NC, M, CH), jnp.float32),
    mesh=_sc_mesh,
    scratch_types=(
        [pltpu.VMEM_SHARED((M, CH), jnp.float32),
         pltpu.VMEM((CHUNKS, EK), jnp.int32),
         pltpu.VMEM((CHUNKS, EK), jnp.int32)]
        + [pltpu.VMEM((EK, CH), jnp.float32)] * (2 * NB)
        + [pltpu.VMEM((EK, 16), jnp.float32)] * NB
        + [pltpu.SemaphoreType.DMA] * (3 * NB)
    ),
)


def _spmm_add(x, init, rows, cols, vals):
  return _spmm_call(x, init, rows, cols, vals)


_PBM = 1000


def _proj0_body(x_ref, w_ref, p1_ref, p2_ref, p3_ref, p4_ref):
  prefs = [p1_ref, p2_ref, p3_ref, p4_ref]
  w = w_ref[...]
  for b in range(B):
    pb = jnp.dot(x_ref[b], w, preferred_element_type=jnp.float32)
    ci, ii = b // 2, b % 2
    for j in range(4):
      prefs[j][ci, :, ii * HID:(ii + 1) * HID] = pb[:, j * HID:(j + 1) * HID]


def _proj0(x, wstack):
  out = jax.ShapeDtypeStruct((NC, M, CH), jnp.float32)
  pspec = pl.BlockSpec((NC, _PBM, CH), lambda i: (0, i, 0))
  return pl.pallas_call(
      _proj0_body,
      grid=(M // _PBM,),
      in_specs=[
          pl.BlockSpec((B, _PBM, W_IN), lambda i: (0, i, 0)),
          pl.BlockSpec((W_IN, 4 * HID), lambda i: (0, 0)),
      ],
      out_specs=[pspec, pspec, pspec, pspec],
      out_shape=[out, out, out, out],
  )(x, wstack)


def _bn_stats(y2):
  n = float(NC * M * 2)
  s64 = jnp.sum(y2, axis=0)
  q64 = jnp.sum(y2 * y2, axis=0)
  s32 = s64[:HID] + s64[HID:]
  q32 = q64[:HID] + q64[HID:]
  mean = s32 / n
  var = q32 / n - mean * mean
  return mean, var


def _bn_scale_off(mean, var, gamma, beta):
  inv = gamma / jnp.sqrt(var + 1e-5)
  scale = jnp.concatenate([inv, inv])
  off = jnp.concatenate([beta - mean * inv, beta - mean * inv])
  return scale, off


def _bn_proj1_body(y_ref, g_ref, b_ref, w_ref, p1_ref, p2_ref, p3_ref, p4_ref):
  y2 = y_ref[...].reshape(NC * M, CH)
  mean, var = _bn_stats(y2)
  scale, off = _bn_scale_off(mean, var, g_ref[0], b_ref[0])
  act = jnp.maximum(y2 * scale[None, :] + off[None, :], 0.0)
  pall = jnp.dot(act, w_ref[...], preferred_element_type=jnp.float32)
  pall = pall.reshape(NC, M, 4 * CH)
  prefs = [p1_ref, p2_ref, p3_ref, p4_ref]
  for j in range(4):
    prefs[j][...] = pall[:, :, j * CH:(j + 1) * CH]


def _bn_proj1(y, gamma, beta, w64):
  out = jax.ShapeDtypeStruct((NC, M, CH), jnp.float32)
  return pl.pallas_call(
      _bn_proj1_body,
      out_shape=[out, out, out, out],
  )(y, gamma.reshape(1, HID), beta.reshape(1, HID), w64)


def _bn_act_body(y_ref, g_ref, b_ref, a_ref):
  y2 = y_ref[...].reshape(NC * M, CH)
  mean, var = _bn_stats(y2)
  scale, off = _bn_scale_off(mean, var, g_ref[0], b_ref[0])
  act = jnp.maximum(y2 * scale[None, :] + off[None, :], 0.0)
  a_ref[...] = act.reshape(NC, M, CH)


def _bn_act(y, gamma, beta):
  return pl.pallas_call(
      _bn_act_body,
      out_shape=jax.ShapeDtypeStruct((NC, M, CH), jnp.float32),
  )(y, gamma.reshape(1, HID), beta.reshape(1, HID))


_FCM = 200
_FCK = _FCM * HID
_FCN = M // _FCM


def _fc_body(a_ref, w_ref, b1_ref, w2_ref, b2_ref, o_ref, acc_ref):
  i = pl.program_id(0)

  @pl.when(i == 0)
  def _():
    acc_ref[...] = jnp.zeros_like(acc_ref)

  a = a_ref[...].reshape(NC, _FCM, 2, HID)
  a = jnp.transpose(a, (0, 2, 1, 3)).reshape(B, _FCK)
  a = jnp.concatenate([a, jnp.zeros((4, _FCK), jnp.float32)], axis=0)
  acc_ref[...] += jnp.dot(a.astype(jnp.bfloat16),
                          w_ref[...].astype(jnp.bfloat16),
                          preferred_element_type=jnp.float32)

  @pl.when(i == _FCN - 1)
  def _():
    h = jnp.maximum(acc_ref[...][:B] + b1_ref[...], 0.0)
    o_ref[...] = jnp.dot(h, w2_ref[...],
                         preferred_element_type=jnp.float32) + b2_ref[...]


def _fc_head(act, fc1_w, fc1_b, fc2_w, fc2_b):
  return pl.pallas_call(
      _fc_body,
      grid=(_FCN,),
      in_specs=[
          pl.BlockSpec((NC, _FCM, CH), lambda i: (0, i, 0)),
          pl.BlockSpec((_FCK, FC1), lambda i: (i, 0)),
          pl.BlockSpec((1, FC1), lambda i: (0, 0)),
          pl.BlockSpec((FC1, NCLS), lambda i: (0, 0)),
          pl.BlockSpec((1, NCLS), lambda i: (0, 0)),
      ],
      out_specs=pl.BlockSpec((B, NCLS), lambda i: (0, 0)),
      out_shape=jax.ShapeDtypeStruct((B, NCLS), jnp.float32),
      scratch_shapes=[pltpu.VMEM((8, FC1), jnp.float32)],
  )(act, fc1_w, fc1_b.reshape(1, FC1), fc2_w, fc2_b.reshape(1, NCLS))


@jax.jit
def kernel(inputs, L_rows, L_cols, L_vals,
           W1_l0, b1_l0, W2_l0, b2_l0, W3_l0, b3_l0, W4_l0, b4_l0,
           gamma_l0, beta_l0,
           W1_l1, b1_l1, W2_l1, b2_l1, W3_l1, b3_l1, W4_l1, b4_l1,
           gamma_l1, beta_l1,
           fc1_W, fc1_b, fc2_W, fc2_b):
  pad = NNZ_PAD - NNZ
  spread = (jnp.arange(pad, dtype=jnp.int32) * 7) % M
  rows = jnp.concatenate([L_rows.astype(jnp.int32), spread])
  rows = rows.reshape(NS, CHUNKS, EK)
  cols = jnp.concatenate([L_cols.astype(jnp.int32), spread])
  cols = cols.reshape(NS, CHUNKS, EK)
  vals = jnp.concatenate([L_vals, jnp.zeros((pad,), jnp.float32)])
  vals = jnp.broadcast_to(vals.reshape(NS, CHUNKS, EK, 1),
                          (NS, CHUNKS, EK, 16))
  vals = jnp.asarray(vals)

  wstack0 = jnp.concatenate(
      [W1_l0 - W3_l0, W2_l0 - 2.0 * W4_l0, W3_l0, W4_l0], axis=1)
  wstack1 = jnp.concatenate(
      [W1_l1 - W3_l1, W2_l1 - 2.0 * W4_l1, W3_l1, W4_l1], axis=1)
  z32 = jnp.zeros((HID, HID), jnp.float32)
  w64 = jnp.concatenate(
      [jnp.block([[wstack1[:, j * HID:(j + 1) * HID], z32],
                  [z32, wstack1[:, j * HID:(j + 1) * HID]]])
       for j in range(4)], axis=1)

  p1, p2, p3, p4 = _proj0(inputs, wstack0)
  t = _spmm_add(p4, p3, rows, cols, vals)
  t = _spmm_add(t, p2, rows, cols, vals)
  y0 = _spmm_add(t, p1, rows, cols, vals)

  q1, q2, q3, q4 = _bn_proj1(y0, gamma_l0, beta_l0, w64)
  t = _spmm_add(q4, q3, rows, cols, vals)
  t = _spmm_add(t, q2, rows, cols, vals)
  y1 = _spmm_add(t, q1, rows, cols, vals)

  act = _bn_act(y1, gamma_l1, beta_l1)
  return _fc_head(act, fc1_W, fc1_b, fc2_W, fc2_b)


# speedup vs baseline: 10.8791x; 1.0053x over previous
import dataclasses
import functools

import jax
import jax.numpy as jnp
from jax import lax
from jax.experimental import pallas as pl
from jax.experimental.pallas import tpu as pltpu
from jax.experimental.pallas import tpu_sc as plsc

M = 10000
W_IN = 128
NNZ = M * 16
B = 4
HID = 32
FC1 = 256
NCLS = 53

NC = 2
NS = 16
CH = (B * HID) // NC
EK = 128
NB = 3
CHUNKS = 81
EPW = CHUNKS * EK
NNZ_PAD = EPW * NS
RPT = 632
RPT_LAST = M - (NS - 1) * RPT

_sc_mesh = plsc.VectorSubcoreMesh(core_axis_name="c", subcore_axis_name="s")


def _spmm_body(x_hbm, init_hbm, rows_hbm, cols_hbm, vals_hbm, out_hbm,
               acc_sh, rows_v, cols_v,
               g0, g1, g2, s0, s1, s2, v0, v1, v2,
               gsem0, gsem1, gsem2, ssem0, ssem1, ssem2,
               vsem0, vsem1, vsem2):
  c = lax.axis_index("c")
  s = lax.axis_index("s")
  gbuf = [g0, g1, g2]
  sbuf = [s0, s1, s2]
  vbuf = [v0, v1, v2]
  gsem = [gsem0, gsem1, gsem2]
  ssem = [ssem0, ssem1, ssem2]
  vsem = [vsem0, vsem1, vsem2]

  row0 = pl.multiple_of(s * RPT, 8)

  @pl.when(s < NS - 1)
  def _():
    pltpu.sync_copy(init_hbm.at[c, pl.ds(row0, RPT)],
                    acc_sh.at[pl.ds(row0, RPT)])

  @pl.when(s == NS - 1)
  def _():
    pltpu.sync_copy(init_hbm.at[c, pl.ds((NS - 1) * RPT, RPT_LAST)],
                    acc_sh.at[pl.ds((NS - 1) * RPT, RPT_LAST)])

  pltpu.sync_copy(rows_hbm.at[s], rows_v)
  pltpu.sync_copy(cols_hbm.at[s], cols_v)
  plsc.subcore_barrier()

  for b in range(NB):
    pltpu.async_copy(x_hbm.at[c].at[cols_v.at[b]], gbuf[b], gsem[b])
    pltpu.async_copy(vals_hbm.at[s, b], vbuf[b], vsem[b])

  @pl.loop(0, CHUNKS, step=NB)
  def _edge_chunk(i):
    for b in range(NB):
      ii = i + b
      pltpu.make_async_copy(x_hbm.at[c].at[cols_v.at[ii]],
                            gbuf[b], gsem[b]).wait()
      pltpu.make_async_copy(vals_hbm.at[s, ii], vbuf[b], vsem[b]).wait()
      @pl.when(ii >= NB)
      def _():
        pltpu.make_async_copy(sbuf[b], acc_sh.at[rows_v.at[ii - NB]],
                              ssem[b]).wait()

      gb, sb, vb = gbuf[b], sbuf[b], vbuf[b]

      @plsc.parallel_loop(0, EK, unroll=4)
      def _scale(e):
        bv = vb[e]
        for k in range(CH // 16):
          sb[e, pl.ds(k * 16, 16)] = gb[e, pl.ds(k * 16, 16)] * bv

      @pl.when(ii + NB < CHUNKS)
      def _():
        pltpu.async_copy(x_hbm.at[c].at[cols_v.at[ii + NB]],
                         gbuf[b], gsem[b])
        pltpu.async_copy(vals_hbm.at[s, ii + NB], vbuf[b], vsem[b])
      pltpu.async_copy(sbuf[b], acc_sh.at[rows_v.at[ii]], ssem[b], add=True)

  for b in range(NB):
    pltpu.make_async_copy(sbuf[b], acc_sh.at[rows_v.at[CHUNKS - NB + b]],
                          ssem[b]).wait()
  plsc.subcore_barrier()

  @pl.when(s < NS - 1)
  def _():
    pltpu.sync_copy(acc_sh.at[pl.ds(row0, RPT)],
                    out_hbm.at[c, pl.ds(row0, RPT)])

  @pl.when(s == NS - 1)
  def _():
    pltpu.sync_copy(acc_sh.at[pl.ds((NS - 1) * RPT, RPT_LAST)],
                    out_hbm.at[c, pl.ds((NS - 1) * RPT, RPT_LAST)])


_sc_params = pltpu.CompilerParams()
if "needs_layout_passes" in pltpu.CompilerParams.__dataclass_fields__:
  _sc_params = dataclasses.replace(_sc_params, needs_layout_passes=False)
if "use_tc_tiling_on_sc" in pltpu.CompilerParams.__dataclass_fields__:
  _sc_params = dataclasses.replace(_sc_params, use_tc_tiling_on_sc=False)

_spmm_call = pl.kernel(
    _spmm_body,
    compiler_params=_sc_params,
    out_type=jax.ShapeDtypeStruct((NC, M, CH), jnp.float32),
    mesh=_sc_mesh,
    scratch_types=(
        [pltpu.VMEM_SHARED((M, CH), jnp.float32),
         pltpu.VMEM((CHUNKS, EK), jnp.int32),
         pltpu.VMEM((CHUNKS, EK), jnp.int32)]
        + [pltpu.VMEM((EK, CH), jnp.float32)] * (2 * NB)
        + [pltpu.VMEM((EK, 16), jnp.float32)] * NB
        + [pltpu.SemaphoreType.DMA] * (3 * NB)
    ),
)


def _spmm_add(x, init, rows, cols, vals):
  return _spmm_call(x, init, rows, cols, vals)


_PBM = 1000


def _proj0_body(x_ref, w_ref, p1_ref, p2_ref, p3_ref, p4_ref):
  prefs = [p1_ref, p2_ref, p3_ref, p4_ref]
  w = w_ref[...]
  for b in range(B):
    pb = jnp.dot(x_ref[b], w, preferred_element_type=jnp.float32)
    ci, ii = b // 2, b % 2
    for j in range(4):
      prefs[j][ci, :, ii * HID:(ii + 1) * HID] = pb[:, j * HID:(j + 1) * HID]


def _proj0(x, wstack):
  out = jax.ShapeDtypeStruct((NC, M, CH), jnp.float32)
  pspec = pl.BlockSpec((NC, _PBM, CH), lambda i: (0, i, 0))
  return pl.pallas_call(
      _proj0_body,
      grid=(M // _PBM,),
      in_specs=[
          pl.BlockSpec((B, _PBM, W_IN), lambda i: (0, i, 0)),
          pl.BlockSpec((W_IN, 4 * HID), lambda i: (0, 0)),
      ],
      out_specs=[pspec, pspec, pspec, pspec],
      out_shape=[out, out, out, out],
  )(x, wstack)


def _bn_stats(y2):
  n = float(NC * M * 2)
  s64 = jnp.sum(y2, axis=0)
  q64 = jnp.sum(y2 * y2, axis=0)
  s32 = s64[:HID] + s64[HID:]
  q32 = q64[:HID] + q64[HID:]
  mean = s32 / n
  var = q32 / n - mean * mean
  return mean, var


def _bn_scale_off(mean, var, gamma, beta):
  inv = gamma / jnp.sqrt(var + 1e-5)
  scale = jnp.concatenate([inv, inv])
  off = jnp.concatenate([beta - mean * inv, beta - mean * inv])
  return scale, off


def _bn_proj1_body(y_ref, g_ref, b_ref, w_ref, p1_ref, p2_ref, p3_ref, p4_ref):
  y2 = y_ref[...].reshape(NC * M, CH)
  mean, var = _bn_stats(y2)
  scale, off = _bn_scale_off(mean, var, g_ref[0], b_ref[0])
  act = jnp.maximum(y2 * scale[None, :] + off[None, :], 0.0)
  pall = jnp.dot(act, w_ref[...], preferred_element_type=jnp.float32)
  pall = pall.reshape(NC, M, 4 * CH)
  prefs = [p1_ref, p2_ref, p3_ref, p4_ref]
  for j in range(4):
    prefs[j][...] = pall[:, :, j * CH:(j + 1) * CH]


def _bn_proj1(y, gamma, beta, w64):
  out = jax.ShapeDtypeStruct((NC, M, CH), jnp.float32)
  return pl.pallas_call(
      _bn_proj1_body,
      out_shape=[out, out, out, out],
  )(y, gamma.reshape(1, HID), beta.reshape(1, HID), w64)


def _bn_act_body(y_ref, g_ref, b_ref, a_ref):
  y2 = y_ref[...].reshape(NC * M, CH)
  mean, var = _bn_stats(y2)
  scale, off = _bn_scale_off(mean, var, g_ref[0], b_ref[0])
  act = jnp.maximum(y2 * scale[None, :] + off[None, :], 0.0)
  a_ref[...] = act.reshape(NC, M, CH)


def _bn_act(y, gamma, beta):
  return pl.pallas_call(
      _bn_act_body,
      out_shape=jax.ShapeDtypeStruct((NC, M, CH), jnp.float32),
  )(y, gamma.reshape(1, HID), beta.reshape(1, HID))


_FCM = 200
_FCK = _FCM * HID
_FCN = M // _FCM


def _fc_body(a_ref, w_ref, b1_ref, w2_ref, b2_ref, o_ref, acc_ref):
  i = pl.program_id(0)

  @pl.when(i == 0)
  def _():
    acc_ref[...] = jnp.zeros_like(acc_ref)

  a = a_ref[...].reshape(NC, _FCM, 2, HID)
  a = jnp.transpose(a, (0, 2, 1, 3)).reshape(B, _FCK)
  a = jnp.concatenate([a, jnp.zeros((4, _FCK), jnp.float32)], axis=0)
  w = w_ref[...]
  wh = w.astype(jnp.bfloat16)
  wl = (w - wh.astype(jnp.float32)).astype(jnp.bfloat16)
  ah = a.astype(jnp.bfloat16)
  al = (a - ah.astype(jnp.float32)).astype(jnp.bfloat16)
  acc_ref[...] += (
      jnp.dot(ah, wh, preferred_element_type=jnp.float32)
      + jnp.dot(ah, wl, preferred_element_type=jnp.float32)
      + jnp.dot(al, wh, preferred_element_type=jnp.float32))

  @pl.when(i == _FCN - 1)
  def _():
    h = jnp.maximum(acc_ref[...][:B] + b1_ref[...], 0.0)
    o_ref[...] = jnp.dot(h, w2_ref[...],
                         preferred_element_type=jnp.float32) + b2_ref[...]


def _fc_head(act, fc1_w, fc1_b, fc2_w, fc2_b):
  return pl.pallas_call(
      _fc_body,
      grid=(_FCN,),
      in_specs=[
          pl.BlockSpec((NC, _FCM, CH), lambda i: (0, i, 0)),
          pl.BlockSpec((_FCK, FC1), lambda i: (i, 0)),
          pl.BlockSpec((1, FC1), lambda i: (0, 0)),
          pl.BlockSpec((FC1, NCLS), lambda i: (0, 0)),
          pl.BlockSpec((1, NCLS), lambda i: (0, 0)),
      ],
      out_specs=pl.BlockSpec((B, NCLS), lambda i: (0, 0)),
      out_shape=jax.ShapeDtypeStruct((B, NCLS), jnp.float32),
      scratch_shapes=[pltpu.VMEM((8, FC1), jnp.float32)],
  )(act, fc1_w, fc1_b.reshape(1, FC1), fc2_w, fc2_b.reshape(1, NCLS))


@jax.jit
def kernel(inputs, L_rows, L_cols, L_vals,
           W1_l0, b1_l0, W2_l0, b2_l0, W3_l0, b3_l0, W4_l0, b4_l0,
           gamma_l0, beta_l0,
           W1_l1, b1_l1, W2_l1, b2_l1, W3_l1, b3_l1, W4_l1, b4_l1,
           gamma_l1, beta_l1,
           fc1_W, fc1_b, fc2_W, fc2_b):
  pad = NNZ_PAD - NNZ
  spread = (jnp.arange(pad, dtype=jnp.int32) * 7) % M
  rows = jnp.concatenate([L_rows.astype(jnp.int32), spread])
  rows = rows.reshape(NS, CHUNKS, EK)
  cols = jnp.concatenate([L_cols.astype(jnp.int32), spread])
  cols = cols.reshape(NS, CHUNKS, EK)
  vals = jnp.concatenate([L_vals, jnp.zeros((pad,), jnp.float32)])
  vals = jnp.broadcast_to(vals.reshape(NS, CHUNKS, EK, 1),
                          (NS, CHUNKS, EK, 16))
  vals = jnp.asarray(vals)

  wstack0 = jnp.concatenate(
      [W1_l0 - W3_l0, W2_l0 - 2.0 * W4_l0, W3_l0, W4_l0], axis=1)
  wstack1 = jnp.concatenate(
      [W1_l1 - W3_l1, W2_l1 - 2.0 * W4_l1, W3_l1, W4_l1], axis=1)
  z32 = jnp.zeros((HID, HID), jnp.float32)
  w64 = jnp.concatenate(
      [jnp.block([[wstack1[:, j * HID:(j + 1) * HID], z32],
                  [z32, wstack1[:, j * HID:(j + 1) * HID]]])
       for j in range(4)], axis=1)

  p1, p2, p3, p4 = _proj0(inputs, wstack0)
  t = _spmm_add(p4, p3, rows, cols, vals)
  t = _spmm_add(t, p2, rows, cols, vals)
  y0 = _spmm_add(t, p1, rows, cols, vals)

  q1, q2, q3, q4 = _bn_proj1(y0, gamma_l0, beta_l0, w64)
  t = _spmm_add(q4, q3, rows, cols, vals)
  t = _spmm_add(t, q2, rows, cols, vals)
  y1 = _spmm_add(t, q1, rows, cols, vals)

  act = _bn_act(y1, gamma_l1, beta_l1)
  return _fc_head(act, fc1_W, fc1_b, fc2_W, fc2_b)
